# baseline (speedup 1.0000x reference)
import functools

import jax
import jax.numpy as jnp
from jax import lax
from jax.experimental import pallas as pl
from jax.experimental.pallas import tpu as pltpu
from jax.experimental.pallas import tpu_sc as plsc

F32 = jnp.float32

_NC = 2
_NS = 16
_NW = _NC * _NS
_LANE = 128
_CH = 640
_KI = _CH // _LANE


def _sc_gather(t, row3, col3, E):
    nchunk = E // _CH
    mesh = plsc.VectorSubcoreMesh(core_axis_name="c", subcore_axis_name="s")

    @functools.partial(
        pl.kernel,
        out_type=(jax.ShapeDtypeStruct((E, 128), F32),
                  jax.ShapeDtypeStruct((E, 128), F32)),
        mesh=mesh,
        scratch_types=[pltpu.VMEM((_KI, _LANE), jnp.int32),
                       pltpu.VMEM((_KI, _LANE), jnp.int32),
                       pltpu.VMEM((_CH, 128), F32),
                       pltpu.SemaphoreType.DMA],
    )
    def k(t_hbm, row_hbm, col_hbm, xr_hbm, xc_hbm, idxa, idxb, buf, sem):
        wid = lax.axis_index("s") * _NC + lax.axis_index("c")

        @pl.loop(wid, nchunk, step=_NW)
        def _(c):
            off = c * _CH
            pltpu.sync_copy(row_hbm.at[c], idxa)
            pltpu.sync_copy(col_hbm.at[c], idxb)
            copies = []
            for j in range(_KI):
                sl = pl.ds(j * _LANE, _LANE)
                copies.append(pltpu.async_copy(t_hbm.at[idxa.at[j]], buf.at[sl], sem))
            for cp in copies:
                cp.wait()
            pltpu.sync_copy(buf, xr_hbm.at[pl.ds(off, _CH)])
            copies = []
            for j in range(_KI):
                sl = pl.ds(j * _LANE, _LANE)
                copies.append(pltpu.async_copy(t_hbm.at[idxb.at[j]], buf.at[sl], sem))
            for cp in copies:
                cp.wait()
            pltpu.sync_copy(buf, xc_hbm.at[pl.ds(off, _CH)])

    return k(t, row3, col3)


def _sc_scatter(upd_e, col3, zeros, N, E):
    half = N // 2
    HP = half + 8
    nchunk = E // _CH
    mesh = plsc.VectorSubcoreMesh(core_axis_name="c", subcore_axis_name="s")

    @functools.partial(
        pl.kernel,
        out_type=jax.ShapeDtypeStruct((_NC, HP, 128), F32),
        mesh=mesh,
        scratch_types=[pltpu.VMEM((_KI, _LANE), jnp.int32),
                       pltpu.VMEM((_KI, _LANE), jnp.int32)]
        + [pltpu.VMEM((_LANE // 8, 128), F32) for _ in range(_KI)]
        + [pltpu.VMEM((_LANE, 128), F32), pltpu.VMEM_SHARED((HP, 128), F32)],
    )
    def k(ue_hbm, col_hbm, z_hbm, out_hbm, *scr):
        idx2 = scr[0]
        idxm = scr[1]
        dats = scr[2:2 + _KI]
        pack = scr[2 + _KI]
        acc = scr[3 + _KI]
        cid = lax.axis_index("c")
        sid = lax.axis_index("s")
        base_node = cid * half

        pltpu.sync_copy(z_hbm.at[pl.ds(0, _LANE)], pack)

        @pl.when(sid == 0)
        def _():
            pltpu.sync_copy(z_hbm, acc)

        plsc.subcore_barrier()

        @pl.loop(sid, nchunk, step=_NS)
        def _(c):
            pltpu.sync_copy(col_hbm.at[c], idx2)
            for j in range(_KI):
                base8 = pl.multiple_of(c * (_CH // 8) + j * (_LANE // 8), _LANE // 8)
                pltpu.sync_copy(ue_hbm.at[pl.ds(base8, _LANE // 8)], dats[j])
            for j in range(_KI):
                for r in range(_LANE // 16):
                    v = idx2[j, pl.ds(r * 16, 16)]
                    inb = (v >= base_node) & (v < base_node + half)
                    idxm[j, pl.ds(r * 16, 16)] = jnp.where(inb, v - base_node, half)
            for j in range(_KI):
                @pl.loop(0, _LANE // 8)
                def _(r):
                    for l in range(8):
                        pack[r * 8 + l, pl.ds(0, 16)] = dats[j][r, pl.ds(l * 16, 16)]
                pltpu.sync_copy(pack, acc.at[idxm.at[j]], add=True)

        plsc.subcore_barrier()

        @pl.when(sid == 0)
        def _():
            pltpu.sync_copy(acc, out_hbm.at[cid])

    return k(upd_e, col3, zeros)


def _relu(v):
    return jnp.maximum(v, 0.0)


def _dot(a, b):
    return jnp.dot(a, b, preferred_element_type=F32)


def _pre0_body(x_ref, wab_ref, t_ref):
    t_ref[...] = _dot(x_ref[...], wab_ref[...])


def _tc_pre0(x, wab):
    N = x.shape[0]
    return pl.pallas_call(
        _pre0_body,
        out_shape=jax.ShapeDtypeStruct((N, 128), F32),
    )(x, wab)


def _edge_tail_body(n_ea, *refs):
    xr, xc = refs[0], refs[1]
    eas = refs[2:2 + n_ea]
    wcs = refs[2 + n_ea:2 + 2 * n_ea]
    b1, w2, b2, w3, b3 = refs[2 + 2 * n_ea:7 + 2 * n_ea]
    out = refs[-1]
    h = xr[:, 0:64] + xc[:, 64:128] + b1[...]
    for ea, wc in zip(eas, wcs):
        h = h + _dot(ea[...], wc[...])
    h = _relu(h)
    h = _relu(_dot(h, w2[...]) + b2[...])
    out[...] = _dot(h, w3[...]) + b3[...]


def _tc_edge_tail(xr, xc, eas, wcs, b1, w2, b2, w3, b3, block_e=8000):
    E = xr.shape[0]
    n_ea = len(eas)
    grid = (E // block_e,)
    eb = lambda i: (i, 0)
    zb = lambda i: (0, 0)
    in_specs = (
        [pl.BlockSpec((block_e, 128), eb), pl.BlockSpec((block_e, 128), eb)]
        + [pl.BlockSpec((block_e, ea.shape[1]), eb) for ea in eas]
        + [pl.BlockSpec(w.shape, zb) for w in wcs]
        + [pl.BlockSpec(b1.shape, zb), pl.BlockSpec(w2.shape, zb),
           pl.BlockSpec(b2.shape, zb), pl.BlockSpec(w3.shape, zb),
           pl.BlockSpec(b3.shape, zb)]
    )
    return pl.pallas_call(
        functools.partial(_edge_tail_body, n_ea),
        grid=grid,
        in_specs=in_specs,
        out_specs=pl.BlockSpec((block_e, 16), eb),
        out_shape=jax.ShapeDtypeStruct((E, 16), F32),
    )(xr, xc, *eas, *wcs, b1, w2, b2, w3, b3)


def _node0_body(x_ref, mp_ref, batch_ref, a1x_ref, a1m_ref, a1b_ref,
                a2_ref, a2b_ref, a3_ref, a3b_ref,
                g1n_ref, g1e_ref, g1b_ref, g2_ref, g2b_ref, g3_ref, g3b_ref,
                vd1_ref, b1u_ref,
                updx_ref, ue1_ref, un1_ref):
    half = mp_ref.shape[1] - 8
    msg = jnp.concatenate([mp_ref[0][0:half, 0:16], mp_ref[1][0:half, 0:16]], axis=0)
    x = x_ref[...]
    h = _relu(_dot(x, a1x_ref[...]) + _dot(msg, a1m_ref[...]) + a1b_ref[...])
    h = _relu(_dot(h, a2_ref[...]) + a2b_ref[...])
    updx = _dot(h, a3_ref[...]) + a3b_ref[...]
    updx_ref[...] = updx
    oh = (batch_ref[...] == lax.broadcasted_iota(jnp.int32, (1, 8), 1)).astype(F32)
    agg_n = _dot(oh.T, updx)
    agg_e = _dot(oh.T, msg)
    g = _relu(_dot(agg_n, g1n_ref[...]) + _dot(agg_e, g1e_ref[...]) + g1b_ref[...])
    g = _relu(_dot(g, g2_ref[...]) + g2b_ref[...])
    updu = _dot(g, g3_ref[...]) + g3b_ref[...]
    ue1_ref[...] = _dot(updu, vd1_ref[...])
    un1_ref[...] = _dot(updu, b1u_ref[...])


def _tc_node0(x, mp, batch2, node_w, glob_w, vd1, b1u):
    N = x.shape[0]
    (a1x, a1m, a1b), (a2, a2b), (a3, a3b) = node_w
    (g1n, g1e, g1b), (g2, g2b), (g3, g3b) = glob_w
    return pl.pallas_call(
        _node0_body,
        out_shape=(jax.ShapeDtypeStruct((N, 128), F32),
                   jax.ShapeDtypeStruct((8, 64), F32),
                   jax.ShapeDtypeStruct((8, 64), F32)),
    )(x, mp, batch2, a1x, a1m, a1b, a2, a2b, a3, a3b,
      g1n, g1e, g1b, g2, g2b, g3, g3b, vd1, b1u)


def _pre1_body(ux_ref, x_ref, batch_ref, ue1p_ref, wu_ref, wx_ref, t_ref):
    oh = (batch_ref[...] == lax.broadcasted_iota(jnp.int32, (1, 8), 1)).astype(F32)
    t_ref[...] = (_dot(ux_ref[...], wu_ref[...]) + _dot(x_ref[...], wx_ref[...])
                  + _dot(oh, ue1p_ref[...]))


def _tc_pre1(ux, x, batch2, ue1p, wu, wx):
    N = x.shape[0]
    return pl.pallas_call(
        _pre1_body,
        out_shape=jax.ShapeDtypeStruct((N, 128), F32),
    )(ux, x, batch2, ue1p, wu, wx)


def _node1_body(ux_ref, x_ref, mp_ref, batch_ref, un1_ref,
                b1a_ref, b1b_ref, b1m_ref, b1bias_ref,
                b2_ref, b2b_ref, b3_ref, b3b_ref,
                wo1_ref, wo2_ref, wo3_ref, bo_ref, out_ref):
    half = mp_ref.shape[1] - 8
    msg = jnp.concatenate([mp_ref[0][0:half, 0:16], mp_ref[1][0:half, 0:16]], axis=0)
    ux = ux_ref[...]
    x = x_ref[...]
    oh = (batch_ref[...] == lax.broadcasted_iota(jnp.int32, (1, 8), 1)).astype(F32)
    h = _relu(_dot(ux, b1a_ref[...]) + _dot(x, b1b_ref[...])
              + _dot(msg, b1m_ref[...]) + _dot(oh, un1_ref[...]) + b1bias_ref[...])
    h = _relu(_dot(h, b2_ref[...]) + b2b_ref[...])
    updx1 = _dot(h, b3_ref[...]) + b3b_ref[...]
    out_ref[...] = (_dot(updx1, wo1_ref[...]) + _dot(ux, wo2_ref[...])
                    + _dot(x, wo3_ref[...]) + bo_ref[...])


def _tc_node1_out(ux, x, mp, batch2, un1, node_w, wo1, wo2, wo3, bo):
    N = x.shape[0]
    (b1a, b1b, b1m, b1bias), (b2, b2b), (b3, b3b) = node_w
    return pl.pallas_call(
        _node1_body,
        out_shape=jax.ShapeDtypeStruct((N, 128), F32),
    )(ux, x, mp, batch2, un1, b1a, b1b, b1m, b1bias,
      b2, b2b, b3, b3b, wo1, wo2, wo3, bo)


def kernel(x, edge_index, edge_attr, batch, params):
    N = x.shape[0]
    E = edge_index.shape[1]
    x0 = x.reshape(N, x.shape[-1])
    ea0 = edge_attr.reshape(E, edge_attr.shape[-1])
    row3 = edge_index[0].reshape(E // _CH, _KI, _LANE)
    col3 = edge_index[1].reshape(E // _CH, _KI, _LANE)
    batch2 = batch.reshape(N, 1)
    col3e = edge_index[1].reshape(E // _CH, _KI, _LANE)
    zeros = jnp.zeros((N // 2 + 8, 128), F32)

    def row2(v):
        return v.reshape(1, -1)

    (W1, b1), (W2, b2), (W3, b3) = params['edge0']
    W1a, W1b, W1c = W1[:128], W1[128:256], W1[256:272]
    (A1, a1b), (A2, a2b), (A3, a3b) = params['node0']
    A1x, A1m = A1[:128], A1[128:144]
    (G1, g1b), (G2, g2b), (G3, g3b) = params['glob0']
    G1n, G1e = G1[:128], G1[128:144]
    (V1, v1b), (V2, v2b), (V3, v3b) = params['edge1']
    Va1, Va2 = V1[0:128], V1[128:256]
    Vb1, Vb2 = V1[256:384], V1[384:512]
    Vc1, Vc2 = V1[512:528], V1[528:544]
    Vd1 = V1[544:560]
    (B1, b1nb), (B2, b2nb), (B3, b3nb) = params['node1']
    B1a, B1b, B1m, B1u = B1[0:128], B1[128:256], B1[256:272], B1[272:288]
    Wo, bo = params['out'][0]
    Wo1, Wo2, Wo3 = Wo[0:128], Wo[128:256], Wo[256:384]

    T0 = _tc_pre0(x0, jnp.concatenate([W1a, W1b], axis=1))
    Xr0, Xc0 = _sc_gather(T0, row3, col3, E)
    upd_e0 = _tc_edge_tail(Xr0, Xc0, [ea0], [W1c],
                           row2(b1), W2, row2(b2), W3, row2(b3))
    mp0 = _sc_scatter(upd_e0.reshape(E // 8, 128), col3e, zeros, N, E)
    upd_x0, Ue1, Un1 = _tc_node0(
        x0, mp0, batch2,
        ((A1x, A1m, row2(a1b)), (A2, row2(a2b)), (A3, row2(a3b))),
        ((G1n, G1e, row2(g1b)), (G2, row2(g2b)), (G3, row2(g3b))),
        Vd1, B1u)

    T1 = _tc_pre1(upd_x0, x0, batch2,
                  jnp.pad(Ue1, ((0, 0), (0, 64))),
                  jnp.concatenate([Va1, Vb1], axis=1),
                  jnp.concatenate([Va2, Vb2], axis=1))
    Xr1, Xc1 = _sc_gather(T1, row3, col3, E)
    upd_e1 = _tc_edge_tail(Xr1, Xc1, [upd_e0, ea0], [Vc1, Vc2],
                           row2(v1b), V2, row2(v2b), V3, row2(v3b))
    mp1 = _sc_scatter(upd_e1.reshape(E // 8, 128), col3e, zeros, N, E)
    out = _tc_node1_out(
        upd_x0, x0, mp1, batch2, Un1,
        ((B1a, B1b, B1m, row2(b1nb)), (B2, row2(b2nb)), (B3, row2(b3nb))),
        Wo1, Wo2, Wo3, row2(bo))

    return out.reshape(N, 1, 1, 128)


# speedup vs baseline: 1.0763x; 1.0763x over previous
import functools

import jax
import jax.numpy as jnp
from jax import lax
from jax.experimental import pallas as pl
from jax.experimental.pallas import tpu as pltpu
from jax.experimental.pallas import tpu_sc as plsc

F32 = jnp.float32
BF16 = jnp.bfloat16

_NC = 2
_NS = 16
_NW = _NC * _NS
_LANE = 128
_CH = 640
_KI = _CH // _LANE


def _sc_gather(t, row3, col3, E):
    nchunk = E // _CH
    mesh = plsc.VectorSubcoreMesh(core_axis_name="c", subcore_axis_name="s")

    @functools.partial(
        pl.kernel,
        out_type=(jax.ShapeDtypeStruct((E, 128), F32),
                  jax.ShapeDtypeStruct((E, 128), F32)),
        mesh=mesh,
        scratch_types=[pltpu.VMEM((_KI, _LANE), jnp.int32),
                       pltpu.VMEM((_KI, _LANE), jnp.int32),
                       pltpu.VMEM((_CH, 128), F32),
                       pltpu.SemaphoreType.DMA],
    )
    def k(t_hbm, row_hbm, col_hbm, xr_hbm, xc_hbm, idxa, idxb, buf, sem):
        wid = lax.axis_index("s") * _NC + lax.axis_index("c")

        @pl.loop(wid, nchunk, step=_NW)
        def _(c):
            off = c * _CH
            pltpu.sync_copy(row_hbm.at[c], idxa)
            pltpu.sync_copy(col_hbm.at[c], idxb)
            copies = []
            for j in range(_KI):
                sl = pl.ds(j * _LANE, _LANE)
                copies.append(pltpu.async_copy(t_hbm.at[idxa.at[j]], buf.at[sl], sem))
            for cp in copies:
                cp.wait()
            pltpu.sync_copy(buf, xr_hbm.at[pl.ds(off, _CH)])
            copies = []
            for j in range(_KI):
                sl = pl.ds(j * _LANE, _LANE)
                copies.append(pltpu.async_copy(t_hbm.at[idxb.at[j]], buf.at[sl], sem))
            for cp in copies:
                cp.wait()
            pltpu.sync_copy(buf, xc_hbm.at[pl.ds(off, _CH)])

    return k(t, row3, col3)


def _sc_scatter(upd_e, col3, zeros, N, E):
    half = N // 2
    HP = half + 8
    nchunk = E // _CH
    mesh = plsc.VectorSubcoreMesh(core_axis_name="c", subcore_axis_name="s")

    @functools.partial(
        pl.kernel,
        out_type=jax.ShapeDtypeStruct((_NC, HP, 128), F32),
        mesh=mesh,
        scratch_types=[pltpu.VMEM((_KI, _LANE), jnp.int32),
                       pltpu.VMEM((_KI, _LANE), jnp.int32)]
        + [pltpu.VMEM((_LANE // 8, 128), F32) for _ in range(_KI)]
        + [pltpu.VMEM((_LANE, 128), F32), pltpu.VMEM((_LANE, 128), F32),
           pltpu.VMEM_SHARED((HP, 128), F32),
           pltpu.SemaphoreType.DMA, pltpu.SemaphoreType.DMA],
    )
    def k(ue_hbm, col_hbm, z_hbm, out_hbm, *scr):
        idx2 = scr[0]
        idxm = scr[1]
        dats = scr[2:2 + _KI]
        packs = scr[2 + _KI:4 + _KI]
        acc = scr[4 + _KI]
        seml, sems = scr[5 + _KI], scr[6 + _KI]
        cid = lax.axis_index("c")
        sid = lax.axis_index("s")
        base_node = cid * half

        pltpu.sync_copy(z_hbm.at[pl.ds(0, _LANE)], packs[0])
        pltpu.sync_copy(z_hbm.at[pl.ds(0, _LANE)], packs[1])

        @pl.when(sid == 0)
        def _():
            pltpu.sync_copy(z_hbm, acc)

        plsc.subcore_barrier()

        @pl.loop(sid, nchunk, step=_NS)
        def _(c):
            loads = [pltpu.async_copy(col_hbm.at[c], idx2, seml)]
            for j in range(_KI):
                base8 = pl.multiple_of(c * (_CH // 8) + j * (_LANE // 8), _LANE // 8)
                loads.append(pltpu.async_copy(
                    ue_hbm.at[pl.ds(base8, _LANE // 8)], dats[j], seml))
            for cp in loads:
                cp.wait()
            for j in range(_KI):
                for r in range(_LANE // 16):
                    v = idx2[j, pl.ds(r * 16, 16)]
                    inb = (v >= base_node) & (v < base_node + half)
                    idxm[j, pl.ds(r * 16, 16)] = jnp.where(inb, v - base_node, half)
            hs = [None, None]
            for j in range(_KI):
                b = j & 1
                if hs[b] is not None:
                    hs[b].wait()
                pk = packs[b]
                for r in range(_LANE // 8):
                    for l in range(8):
                        pk[r * 8 + l, pl.ds(0, 16)] = dats[j][r, pl.ds(l * 16, 16)]
                hs[b] = pltpu.async_copy(pk, acc.at[idxm.at[j]], sems, add=True)
            for h in hs:
                if h is not None:
                    h.wait()

        plsc.subcore_barrier()

        @pl.when(sid == 0)
        def _():
            pltpu.sync_copy(acc, out_hbm.at[cid])

    return k(upd_e, col3, zeros)


def _relu(v):
    return jnp.maximum(v, 0.0)


def _dot(a, b):
    return jnp.dot(a, b, preferred_element_type=F32)


def _pre0_body(x_ref, wab_ref, t_ref):
    t_ref[...] = _dot(x_ref[...], wab_ref[...])


def _tc_pre0(x, wab):
    N = x.shape[0]
    return pl.pallas_call(
        _pre0_body,
        out_shape=jax.ShapeDtypeStruct((N, 128), F32),
    )(x, wab)


def _edge_tail_body(n_ea, *refs):
    xr, xc = refs[0], refs[1]
    eas = refs[2:2 + n_ea]
    wcs = refs[2 + n_ea:2 + 2 * n_ea]
    b1, w2, b2, w3, b3 = refs[2 + 2 * n_ea:7 + 2 * n_ea]
    out = refs[-1]
    h = xr[:, 0:64] + xc[:, 64:128] + b1[...]
    for ea, wc in zip(eas, wcs):
        h = h + _dot(ea[...], wc[...])
    h = _relu(h)
    h = _relu(_dot(h, w2[...]) + b2[...])
    out[...] = _dot(h, w3[...]) + b3[...]


def _tc_edge_tail(xr, xc, eas, wcs, b1, w2, b2, w3, b3, block_e=8000):
    E = xr.shape[0]
    n_ea = len(eas)
    grid = (E // block_e,)
    eb = lambda i: (i, 0)
    zb = lambda i: (0, 0)
    in_specs = (
        [pl.BlockSpec((block_e, 128), eb), pl.BlockSpec((block_e, 128), eb)]
        + [pl.BlockSpec((block_e, ea.shape[1]), eb) for ea in eas]
        + [pl.BlockSpec(w.shape, zb) for w in wcs]
        + [pl.BlockSpec(b1.shape, zb), pl.BlockSpec(w2.shape, zb),
           pl.BlockSpec(b2.shape, zb), pl.BlockSpec(w3.shape, zb),
           pl.BlockSpec(b3.shape, zb)]
    )
    return pl.pallas_call(
        functools.partial(_edge_tail_body, n_ea),
        grid=grid,
        in_specs=in_specs,
        out_specs=pl.BlockSpec((block_e, 16), eb),
        out_shape=jax.ShapeDtypeStruct((E, 16), F32),
    )(xr, xc, *eas, *wcs, b1, w2, b2, w3, b3)


def _node0_body(x_ref, mp_ref, batch_ref, a1x_ref, a1m_ref, a1b_ref,
                a2_ref, a2b_ref, a3_ref, a3b_ref,
                g1n_ref, g1e_ref, g1b_ref, g2_ref, g2b_ref, g3_ref, g3b_ref,
                vd1_ref, b1u_ref,
                updx_ref, ue1_ref, un1_ref):
    half = mp_ref.shape[1] - 8
    msg = jnp.concatenate([mp_ref[0][0:half, 0:16], mp_ref[1][0:half, 0:16]], axis=0)
    x = x_ref[...]
    h = _relu(_dot(x, a1x_ref[...]) + _dot(msg, a1m_ref[...]) + a1b_ref[...])
    h = _relu(_dot(h, a2_ref[...]) + a2b_ref[...])
    updx = _dot(h, a3_ref[...]) + a3b_ref[...]
    updx_ref[...] = updx
    oh = (batch_ref[...] == lax.broadcasted_iota(jnp.int32, (1, 8), 1)).astype(F32)
    agg_n = _dot(oh.T, updx)
    agg_e = _dot(oh.T, msg)
    g = _relu(_dot(agg_n, g1n_ref[...]) + _dot(agg_e, g1e_ref[...]) + g1b_ref[...])
    g = _relu(_dot(g, g2_ref[...]) + g2b_ref[...])
    updu = _dot(g, g3_ref[...]) + g3b_ref[...]
    ue1_ref[...] = _dot(updu, vd1_ref[...])
    un1_ref[...] = _dot(updu, b1u_ref[...])


def _tc_node0(x, mp, batch2, node_w, glob_w, vd1, b1u):
    N = x.shape[0]
    (a1x, a1m, a1b), (a2, a2b), (a3, a3b) = node_w
    (g1n, g1e, g1b), (g2, g2b), (g3, g3b) = glob_w
    return pl.pallas_call(
        _node0_body,
        out_shape=(jax.ShapeDtypeStruct((N, 128), F32),
                   jax.ShapeDtypeStruct((8, 64), F32),
                   jax.ShapeDtypeStruct((8, 64), F32)),
    )(x, mp, batch2, a1x, a1m, a1b, a2, a2b, a3, a3b,
      g1n, g1e, g1b, g2, g2b, g3, g3b, vd1, b1u)


def _pre1_body(ux_ref, x_ref, batch_ref, ue1p_ref, wu_ref, wx_ref, t_ref):
    oh = (batch_ref[...] == lax.broadcasted_iota(jnp.int32, (1, 8), 1)).astype(F32)
    t_ref[...] = (_dot(ux_ref[...], wu_ref[...]) + _dot(x_ref[...], wx_ref[...])
                  + _dot(oh, ue1p_ref[...]))


def _tc_pre1(ux, x, batch2, ue1p, wu, wx):
    N = x.shape[0]
    return pl.pallas_call(
        _pre1_body,
        out_shape=jax.ShapeDtypeStruct((N, 128), F32),
    )(ux, x, batch2, ue1p, wu, wx)


def _node1_body(ux_ref, x_ref, mp_ref, batch_ref, un1_ref,
                b1a_ref, b1b_ref, b1m_ref, b1bias_ref,
                b2_ref, b2b_ref, b3_ref, b3b_ref,
                wo1_ref, wo2_ref, wo3_ref, bo_ref, out_ref):
    half = mp_ref.shape[1] - 8
    msg = jnp.concatenate([mp_ref[0][0:half, 0:16], mp_ref[1][0:half, 0:16]], axis=0)
    ux = ux_ref[...]
    x = x_ref[...]
    oh = (batch_ref[...] == lax.broadcasted_iota(jnp.int32, (1, 8), 1)).astype(F32)
    h = _relu(_dot(ux, b1a_ref[...]) + _dot(x, b1b_ref[...])
              + _dot(msg, b1m_ref[...]) + _dot(oh, un1_ref[...]) + b1bias_ref[...])
    h = _relu(_dot(h, b2_ref[...]) + b2b_ref[...])
    updx1 = _dot(h, b3_ref[...]) + b3b_ref[...]
    out_ref[...] = (_dot(updx1, wo1_ref[...]) + _dot(ux, wo2_ref[...])
                    + _dot(x, wo3_ref[...]) + bo_ref[...])


def _tc_node1_out(ux, x, mp, batch2, un1, node_w, wo1, wo2, wo3, bo):
    N = x.shape[0]
    (b1a, b1b, b1m, b1bias), (b2, b2b), (b3, b3b) = node_w
    return pl.pallas_call(
        _node1_body,
        out_shape=jax.ShapeDtypeStruct((N, 128), F32),
    )(ux, x, mp, batch2, un1, b1a, b1b, b1m, b1bias,
      b2, b2b, b3, b3b, wo1, wo2, wo3, bo)


def kernel(x, edge_index, edge_attr, batch, params):
    N = x.shape[0]
    E = edge_index.shape[1]
    x0 = x.reshape(N, x.shape[-1])
    ea0 = edge_attr.reshape(E, edge_attr.shape[-1])
    row3 = edge_index[0].reshape(E // _CH, _KI, _LANE)
    col3 = edge_index[1].reshape(E // _CH, _KI, _LANE)
    batch2 = batch.reshape(N, 1)
    col3e = edge_index[1].reshape(E // _CH, _KI, _LANE)
    zeros = jnp.zeros((N // 2 + 8, 128), F32)

    def row2(v):
        return v.reshape(1, -1)

    (W1, b1), (W2, b2), (W3, b3) = params['edge0']
    W1a, W1b, W1c = W1[:128], W1[128:256], W1[256:272]
    (A1, a1b), (A2, a2b), (A3, a3b) = params['node0']
    A1x, A1m = A1[:128], A1[128:144]
    (G1, g1b), (G2, g2b), (G3, g3b) = params['glob0']
    G1n, G1e = G1[:128], G1[128:144]
    (V1, v1b), (V2, v2b), (V3, v3b) = params['edge1']
    Va1, Va2 = V1[0:128], V1[128:256]
    Vb1, Vb2 = V1[256:384], V1[384:512]
    Vc1, Vc2 = V1[512:528], V1[528:544]
    Vd1 = V1[544:560]
    (B1, b1nb), (B2, b2nb), (B3, b3nb) = params['node1']
    B1a, B1b, B1m, B1u = B1[0:128], B1[128:256], B1[256:272], B1[272:288]
    Wo, bo = params['out'][0]
    Wo1, Wo2, Wo3 = Wo[0:128], Wo[128:256], Wo[256:384]

    T0 = _tc_pre0(x0, jnp.concatenate([W1a, W1b], axis=1))
    Xr0, Xc0 = _sc_gather(T0, row3, col3, E)
    upd_e0 = _tc_edge_tail(Xr0, Xc0, [ea0], [W1c],
                           row2(b1), W2, row2(b2), W3, row2(b3))
    mp0 = _sc_scatter(upd_e0.reshape(E // 8, 128), col3e, zeros, N, E)
    upd_x0, Ue1, Un1 = _tc_node0(
        x0, mp0, batch2,
        ((A1x, A1m, row2(a1b)), (A2, row2(a2b)), (A3, row2(a3b))),
        ((G1n, G1e, row2(g1b)), (G2, row2(g2b)), (G3, row2(g3b))),
        Vd1, B1u)

    T1 = _tc_pre1(upd_x0, x0, batch2,
                  jnp.pad(Ue1, ((0, 0), (0, 64))),
                  jnp.concatenate([Va1, Vb1], axis=1),
                  jnp.concatenate([Va2, Vb2], axis=1))
    Xr1, Xc1 = _sc_gather(T1, row3, col3, E)
    upd_e1 = _tc_edge_tail(Xr1, Xc1, [upd_e0, ea0], [Vc1, Vc2],
                           row2(v1b), V2, row2(v2b), V3, row2(v3b))
    mp1 = _sc_scatter(upd_e1.reshape(E // 8, 128), col3e, zeros, N, E)
    out = _tc_node1_out(
        upd_x0, x0, mp1, batch2, Un1,
        ((B1a, B1b, B1m, row2(b1nb)), (B2, row2(b2nb)), (B3, row2(b3nb))),
        Wo1, Wo2, Wo3, row2(bo))

    return out.reshape(N, 1, 1, 128)


# speedup vs baseline: 1.0830x; 1.0062x over previous
import functools

import jax
import jax.numpy as jnp
from jax import lax
from jax.experimental import pallas as pl
from jax.experimental.pallas import tpu as pltpu
from jax.experimental.pallas import tpu_sc as plsc

F32 = jnp.float32
BF16 = jnp.bfloat16

_NC = 2
_NS = 16
_NW = _NC * _NS
_LANE = 128
_CH = 640
_KI = _CH // _LANE


def _sc_gather(t, row3, col3, E):
    nchunk = E // _CH
    mesh = plsc.VectorSubcoreMesh(core_axis_name="c", subcore_axis_name="s")

    @functools.partial(
        pl.kernel,
        out_type=(jax.ShapeDtypeStruct((E, 128), F32),
                  jax.ShapeDtypeStruct((E, 128), F32)),
        mesh=mesh,
        scratch_types=[pltpu.VMEM((_KI, _LANE), jnp.int32),
                       pltpu.VMEM((_KI, _LANE), jnp.int32),
                       pltpu.VMEM((_CH, 128), F32),
                       pltpu.SemaphoreType.DMA],
    )
    def k(t_hbm, row_hbm, col_hbm, xr_hbm, xc_hbm, idxa, idxb, buf, sem):
        wid = lax.axis_index("s") * _NC + lax.axis_index("c")

        @pl.loop(wid, nchunk, step=_NW)
        def _(c):
            off = c * _CH
            il = [pltpu.async_copy(row_hbm.at[c], idxa, sem),
                  pltpu.async_copy(col_hbm.at[c], idxb, sem)]
            for cp in il:
                cp.wait()
            copies = []
            for j in range(_KI):
                sl = pl.ds(j * _LANE, _LANE)
                copies.append(pltpu.async_copy(t_hbm.at[idxa.at[j]], buf.at[sl], sem))
            for cp in copies:
                cp.wait()
            pltpu.sync_copy(buf, xr_hbm.at[pl.ds(off, _CH)])
            copies = []
            for j in range(_KI):
                sl = pl.ds(j * _LANE, _LANE)
                copies.append(pltpu.async_copy(t_hbm.at[idxb.at[j]], buf.at[sl], sem))
            for cp in copies:
                cp.wait()
            pltpu.sync_copy(buf, xc_hbm.at[pl.ds(off, _CH)])

    return k(t, row3, col3)


def _sc_scatter(upd_e, col3, zeros, N, E):
    half = N // 2
    HP = half + 8
    nchunk = E // _CH
    mesh = plsc.VectorSubcoreMesh(core_axis_name="c", subcore_axis_name="s")

    @functools.partial(
        pl.kernel,
        out_type=jax.ShapeDtypeStruct((_NC, HP, 128), F32),
        mesh=mesh,
        scratch_types=[pltpu.VMEM((_KI, _LANE), jnp.int32),
                       pltpu.VMEM((_KI, _LANE), jnp.int32)]
        + [pltpu.VMEM((_LANE // 8, 128), F32) for _ in range(_KI)]
        + [pltpu.VMEM((_LANE, 128), F32), pltpu.VMEM((_LANE, 128), F32),
           pltpu.VMEM_SHARED((HP, 128), F32),
           pltpu.SemaphoreType.DMA, pltpu.SemaphoreType.DMA],
    )
    def k(ue_hbm, col_hbm, z_hbm, out_hbm, *scr):
        idx2 = scr[0]
        idxm = scr[1]
        dats = scr[2:2 + _KI]
        packs = scr[2 + _KI:4 + _KI]
        acc = scr[4 + _KI]
        seml, sems = scr[5 + _KI], scr[6 + _KI]
        cid = lax.axis_index("c")
        sid = lax.axis_index("s")
        base_node = cid * half

        pltpu.sync_copy(z_hbm.at[pl.ds(0, _LANE)], packs[0])
        pltpu.sync_copy(z_hbm.at[pl.ds(0, _LANE)], packs[1])

        @pl.when(sid == 0)
        def _():
            pltpu.sync_copy(z_hbm, acc)

        plsc.subcore_barrier()

        @pl.loop(sid, nchunk, step=_NS)
        def _(c):
            loads = [pltpu.async_copy(col_hbm.at[c], idx2, seml)]
            for j in range(_KI):
                base8 = pl.multiple_of(c * (_CH // 8) + j * (_LANE // 8), _LANE // 8)
                loads.append(pltpu.async_copy(
                    ue_hbm.at[pl.ds(base8, _LANE // 8)], dats[j], seml))
            for cp in loads:
                cp.wait()
            for j in range(_KI):
                for r in range(_LANE // 16):
                    v = idx2[j, pl.ds(r * 16, 16)]
                    inb = (v >= base_node) & (v < base_node + half)
                    idxm[j, pl.ds(r * 16, 16)] = jnp.where(inb, v - base_node, half)
            hs = [None, None]
            for j in range(_KI):
                b = j & 1
                if hs[b] is not None:
                    hs[b].wait()
                pk = packs[b]
                for r in range(_LANE // 8):
                    for l in range(8):
                        pk[r * 8 + l, pl.ds(0, 16)] = dats[j][r, pl.ds(l * 16, 16)]
                hs[b] = pltpu.async_copy(pk, acc.at[idxm.at[j]], sems, add=True)
            for h in hs:
                if h is not None:
                    h.wait()

        plsc.subcore_barrier()

        @pl.when(sid == 0)
        def _():
            pltpu.sync_copy(acc, out_hbm.at[cid])

    return k(upd_e, col3, zeros)


def _relu(v):
    return jnp.maximum(v, 0.0)


def _dot(a, b):
    return jnp.dot(a, b, preferred_element_type=F32)


def _pre0_body(x_ref, wab_ref, t_ref):
    t_ref[...] = _dot(x_ref[...], wab_ref[...])


def _tc_pre0(x, wab):
    N = x.shape[0]
    return pl.pallas_call(
        _pre0_body,
        out_shape=jax.ShapeDtypeStruct((N, 128), F32),
    )(x, wab)


def _edge_tail_body(n_ea, *refs):
    xr, xc = refs[0], refs[1]
    eas = refs[2:2 + n_ea]
    wcs = refs[2 + n_ea:2 + 2 * n_ea]
    b1, w2, b2, w3, b3 = refs[2 + 2 * n_ea:7 + 2 * n_ea]
    out = refs[-1]
    h = xr[:, 0:64] + xc[:, 64:128] + b1[...]
    for ea, wc in zip(eas, wcs):
        h = h + _dot(ea[...], wc[...])
    h = _relu(h)
    h = _relu(_dot(h, w2[...]) + b2[...])
    out[...] = _dot(h, w3[...]) + b3[...]


def _tc_edge_tail(xr, xc, eas, wcs, b1, w2, b2, w3, b3, block_e=8000):
    E = xr.shape[0]
    n_ea = len(eas)
    grid = (E // block_e,)
    eb = lambda i: (i, 0)
    zb = lambda i: (0, 0)
    in_specs = (
        [pl.BlockSpec((block_e, 128), eb), pl.BlockSpec((block_e, 128), eb)]
        + [pl.BlockSpec((block_e, ea.shape[1]), eb) for ea in eas]
        + [pl.BlockSpec(w.shape, zb) for w in wcs]
        + [pl.BlockSpec(b1.shape, zb), pl.BlockSpec(w2.shape, zb),
           pl.BlockSpec(b2.shape, zb), pl.BlockSpec(w3.shape, zb),
           pl.BlockSpec(b3.shape, zb)]
    )
    return pl.pallas_call(
        functools.partial(_edge_tail_body, n_ea),
        grid=grid,
        in_specs=in_specs,
        out_specs=pl.BlockSpec((block_e, 16), eb),
        out_shape=jax.ShapeDtypeStruct((E, 16), F32),
    )(xr, xc, *eas, *wcs, b1, w2, b2, w3, b3)


def _node0_body(x_ref, mp_ref, batch_ref, a1x_ref, a1m_ref, a1b_ref,
                a2_ref, a2b_ref, a3_ref, a3b_ref,
                g1n_ref, g1e_ref, g1b_ref, g2_ref, g2b_ref, g3_ref, g3b_ref,
                vd1_ref, b1u_ref,
                updx_ref, ue1_ref, un1_ref):
    half = mp_ref.shape[1] - 8
    msg = jnp.concatenate([mp_ref[0][0:half, 0:16], mp_ref[1][0:half, 0:16]], axis=0)
    x = x_ref[...]
    h = _relu(_dot(x, a1x_ref[...]) + _dot(msg, a1m_ref[...]) + a1b_ref[...])
    h = _relu(_dot(h, a2_ref[...]) + a2b_ref[...])
    updx = _dot(h, a3_ref[...]) + a3b_ref[...]
    updx_ref[...] = updx
    oh = (batch_ref[...] == lax.broadcasted_iota(jnp.int32, (1, 8), 1)).astype(F32)
    agg_n = _dot(oh.T, updx)
    agg_e = _dot(oh.T, msg)
    g = _relu(_dot(agg_n, g1n_ref[...]) + _dot(agg_e, g1e_ref[...]) + g1b_ref[...])
    g = _relu(_dot(g, g2_ref[...]) + g2b_ref[...])
    updu = _dot(g, g3_ref[...]) + g3b_ref[...]
    ue1_ref[...] = _dot(updu, vd1_ref[...])
    un1_ref[...] = _dot(updu, b1u_ref[...])


def _tc_node0(x, mp, batch2, node_w, glob_w, vd1, b1u):
    N = x.shape[0]
    (a1x, a1m, a1b), (a2, a2b), (a3, a3b) = node_w
    (g1n, g1e, g1b), (g2, g2b), (g3, g3b) = glob_w
    return pl.pallas_call(
        _node0_body,
        out_shape=(jax.ShapeDtypeStruct((N, 128), F32),
                   jax.ShapeDtypeStruct((8, 64), F32),
                   jax.ShapeDtypeStruct((8, 64), F32)),
    )(x, mp, batch2, a1x, a1m, a1b, a2, a2b, a3, a3b,
      g1n, g1e, g1b, g2, g2b, g3, g3b, vd1, b1u)


def _pre1_body(ux_ref, x_ref, batch_ref, ue1p_ref, wu_ref, wx_ref, t_ref):
    oh = (batch_ref[...] == lax.broadcasted_iota(jnp.int32, (1, 8), 1)).astype(F32)
    t_ref[...] = (_dot(ux_ref[...], wu_ref[...]) + _dot(x_ref[...], wx_ref[...])
                  + _dot(oh, ue1p_ref[...]))


def _tc_pre1(ux, x, batch2, ue1p, wu, wx):
    N = x.shape[0]
    return pl.pallas_call(
        _pre1_body,
        out_shape=jax.ShapeDtypeStruct((N, 128), F32),
    )(ux, x, batch2, ue1p, wu, wx)


def _node1_body(ux_ref, x_ref, mp_ref, batch_ref, un1_ref,
                b1a_ref, b1b_ref, b1m_ref, b1bias_ref,
                b2_ref, b2b_ref, b3_ref, b3b_ref,
                wo1_ref, wo2_ref, wo3_ref, bo_ref, out_ref):
    half = mp_ref.shape[1] - 8
    msg = jnp.concatenate([mp_ref[0][0:half, 0:16], mp_ref[1][0:half, 0:16]], axis=0)
    ux = ux_ref[...]
    x = x_ref[...]
    oh = (batch_ref[...] == lax.broadcasted_iota(jnp.int32, (1, 8), 1)).astype(F32)
    h = _relu(_dot(ux, b1a_ref[...]) + _dot(x, b1b_ref[...])
              + _dot(msg, b1m_ref[...]) + _dot(oh, un1_ref[...]) + b1bias_ref[...])
    h = _relu(_dot(h, b2_ref[...]) + b2b_ref[...])
    updx1 = _dot(h, b3_ref[...]) + b3b_ref[...]
    out_ref[...] = (_dot(updx1, wo1_ref[...]) + _dot(ux, wo2_ref[...])
                    + _dot(x, wo3_ref[...]) + bo_ref[...])


def _tc_node1_out(ux, x, mp, batch2, un1, node_w, wo1, wo2, wo3, bo):
    N = x.shape[0]
    (b1a, b1b, b1m, b1bias), (b2, b2b), (b3, b3b) = node_w
    return pl.pallas_call(
        _node1_body,
        out_shape=jax.ShapeDtypeStruct((N, 128), F32),
    )(ux, x, mp, batch2, un1, b1a, b1b, b1m, b1bias,
      b2, b2b, b3, b3b, wo1, wo2, wo3, bo)


def kernel(x, edge_index, edge_attr, batch, params):
    N = x.shape[0]
    E = edge_index.shape[1]
    x0 = x.reshape(N, x.shape[-1])
    ea0 = edge_attr.reshape(E, edge_attr.shape[-1])
    row3 = edge_index[0].reshape(E // _CH, _KI, _LANE)
    col3 = edge_index[1].reshape(E // _CH, _KI, _LANE)
    batch2 = batch.reshape(N, 1)
    col3e = edge_index[1].reshape(E // _CH, _KI, _LANE)
    zeros = jnp.zeros((N // 2 + 8, 128), F32)

    def row2(v):
        return v.reshape(1, -1)

    (W1, b1), (W2, b2), (W3, b3) = params['edge0']
    W1a, W1b, W1c = W1[:128], W1[128:256], W1[256:272]
    (A1, a1b), (A2, a2b), (A3, a3b) = params['node0']
    A1x, A1m = A1[:128], A1[128:144]
    (G1, g1b), (G2, g2b), (G3, g3b) = params['glob0']
    G1n, G1e = G1[:128], G1[128:144]
    (V1, v1b), (V2, v2b), (V3, v3b) = params['edge1']
    Va1, Va2 = V1[0:128], V1[128:256]
    Vb1, Vb2 = V1[256:384], V1[384:512]
    Vc1, Vc2 = V1[512:528], V1[528:544]
    Vd1 = V1[544:560]
    (B1, b1nb), (B2, b2nb), (B3, b3nb) = params['node1']
    B1a, B1b, B1m, B1u = B1[0:128], B1[128:256], B1[256:272], B1[272:288]
    Wo, bo = params['out'][0]
    Wo1, Wo2, Wo3 = Wo[0:128], Wo[128:256], Wo[256:384]

    T0 = _tc_pre0(x0, jnp.concatenate([W1a, W1b], axis=1))
    Xr0, Xc0 = _sc_gather(T0, row3, col3, E)
    upd_e0 = _tc_edge_tail(Xr0, Xc0, [ea0], [W1c],
                           row2(b1), W2, row2(b2), W3, row2(b3))
    mp0 = _sc_scatter(upd_e0.reshape(E // 8, 128), col3e, zeros, N, E)
    upd_x0, Ue1, Un1 = _tc_node0(
        x0, mp0, batch2,
        ((A1x, A1m, row2(a1b)), (A2, row2(a2b)), (A3, row2(a3b))),
        ((G1n, G1e, row2(g1b)), (G2, row2(g2b)), (G3, row2(g3b))),
        Vd1, B1u)

    T1 = _tc_pre1(upd_x0, x0, batch2,
                  jnp.pad(Ue1, ((0, 0), (0, 64))),
                  jnp.concatenate([Va1, Vb1], axis=1),
                  jnp.concatenate([Va2, Vb2], axis=1))
    Xr1, Xc1 = _sc_gather(T1, row3, col3, E)
    upd_e1 = _tc_edge_tail(Xr1, Xc1, [upd_e0, ea0], [Vc1, Vc2],
                           row2(v1b), V2, row2(v2b), V3, row2(v3b))
    mp1 = _sc_scatter(upd_e1.reshape(E // 8, 128), col3e, zeros, N, E)
    out = _tc_node1_out(
        upd_x0, x0, mp1, batch2, Un1,
        ((B1a, B1b, B1m, row2(b1nb)), (B2, row2(b2nb)), (B3, row2(b3nb))),
        Wo1, Wo2, Wo3, row2(bo))

    return out.reshape(N, 1, 1, 128)


# speedup vs baseline: 1.0906x; 1.0070x over previous
import functools

import jax
import jax.numpy as jnp
from jax import lax
from jax.experimental import pallas as pl
from jax.experimental.pallas import tpu as pltpu
from jax.experimental.pallas import tpu_sc as plsc

F32 = jnp.float32
BF16 = jnp.bfloat16

_NC = 2
_NS = 16
_NW = _NC * _NS
_LANE = 128
_CH = 640
_KI = _CH // _LANE


def _sc_gather(t, row3, col3, E):
    nchunk = E // _CH
    mesh = plsc.VectorSubcoreMesh(core_axis_name="c", subcore_axis_name="s")

    @functools.partial(
        pl.kernel,
        out_type=(jax.ShapeDtypeStruct((E, 128), F32),
                  jax.ShapeDtypeStruct((E, 128), F32)),
        mesh=mesh,
        scratch_types=[pltpu.VMEM((_KI, _LANE), jnp.int32),
                       pltpu.VMEM((_KI, _LANE), jnp.int32),
                       pltpu.VMEM((_CH, 128), F32),
                       pltpu.SemaphoreType.DMA],
    )
    def k(t_hbm, row_hbm, col_hbm, xr_hbm, xc_hbm, idxa, idxb, buf, sem):
        wid = lax.axis_index("s") * _NC + lax.axis_index("c")

        @pl.loop(wid, nchunk, step=_NW)
        def _(c):
            off = c * _CH
            il = [pltpu.async_copy(row_hbm.at[c], idxa, sem),
                  pltpu.async_copy(col_hbm.at[c], idxb, sem)]
            for cp in il:
                cp.wait()
            copies = []
            for j in range(_KI):
                sl = pl.ds(j * _LANE, _LANE)
                copies.append(pltpu.async_copy(t_hbm.at[idxa.at[j]], buf.at[sl], sem))
            for cp in copies:
                cp.wait()
            pltpu.sync_copy(buf, xr_hbm.at[pl.ds(off, _CH)])
            copies = []
            for j in range(_KI):
                sl = pl.ds(j * _LANE, _LANE)
                copies.append(pltpu.async_copy(t_hbm.at[idxb.at[j]], buf.at[sl], sem))
            for cp in copies:
                cp.wait()
            pltpu.sync_copy(buf, xc_hbm.at[pl.ds(off, _CH)])

    return k(t, row3, col3)


def _sc_scatter(upd_e, col3, zeros, N, E):
    half = N // 2
    HP = half + 8
    nchunk = E // _CH
    mesh = plsc.VectorSubcoreMesh(core_axis_name="c", subcore_axis_name="s")

    @functools.partial(
        pl.kernel,
        out_type=jax.ShapeDtypeStruct((_NC, HP, 128), F32),
        mesh=mesh,
        scratch_types=[pltpu.VMEM((_KI, _LANE), jnp.int32),
                       pltpu.VMEM((_KI, _LANE), jnp.int32)]
        + [pltpu.VMEM((_LANE // 8, 128), F32) for _ in range(_KI)]
        + [pltpu.VMEM((_LANE, 128), F32), pltpu.VMEM((_LANE, 128), F32),
           pltpu.VMEM_SHARED((HP, 128), F32),
           pltpu.SemaphoreType.DMA, pltpu.SemaphoreType.DMA],
    )
    def k(ue_hbm, col_hbm, z_hbm, out_hbm, *scr):
        idx2 = scr[0]
        idxm = scr[1]
        dats = scr[2:2 + _KI]
        packs = scr[2 + _KI:4 + _KI]
        acc = scr[4 + _KI]
        seml, sems = scr[5 + _KI], scr[6 + _KI]
        cid = lax.axis_index("c")
        sid = lax.axis_index("s")
        base_node = cid * half

        pltpu.sync_copy(z_hbm.at[pl.ds(0, _LANE)], packs[0])
        pltpu.sync_copy(z_hbm.at[pl.ds(0, _LANE)], packs[1])

        @pl.when(sid == 0)
        def _():
            pltpu.sync_copy(z_hbm, acc)

        plsc.subcore_barrier()

        @pl.loop(sid, nchunk, step=_NS)
        def _(c):
            loads = [pltpu.async_copy(col_hbm.at[c], idx2, seml)]
            for j in range(_KI):
                base8 = pl.multiple_of(c * (_CH // 8) + j * (_LANE // 8), _LANE // 8)
                loads.append(pltpu.async_copy(
                    ue_hbm.at[pl.ds(base8, _LANE // 8)], dats[j], seml))
            for cp in loads:
                cp.wait()
            for j in range(_KI):
                for r in range(_LANE // 16):
                    v = idx2[j, pl.ds(r * 16, 16)]
                    inb = (v >= base_node) & (v < base_node + half)
                    idxm[j, pl.ds(r * 16, 16)] = jnp.where(inb, v - base_node, half)
            hs = [None, None]
            for j in range(_KI):
                b = j & 1
                if hs[b] is not None:
                    hs[b].wait()
                pk = packs[b]
                for r in range(_LANE // 8):
                    for l in range(8):
                        pk[r * 8 + l, pl.ds(0, 16)] = dats[j][r, pl.ds(l * 16, 16)]
                hs[b] = pltpu.async_copy(pk, acc.at[idxm.at[j]], sems, add=True)
            for h in hs:
                if h is not None:
                    h.wait()

        plsc.subcore_barrier()

        @pl.when(sid == 0)
        def _():
            pltpu.sync_copy(acc, out_hbm.at[cid])

    return k(upd_e, col3, zeros)


def _relu(v):
    return jnp.maximum(v, 0.0)


def _dot(a, b):
    return jnp.dot(a, b, preferred_element_type=F32)


def _pre0_body(x_ref, wab_ref, t_ref):
    t_ref[...] = _dot(x_ref[...], wab_ref[...])


def _tc_pre0(x, wab):
    N = x.shape[0]
    return pl.pallas_call(
        _pre0_body,
        out_shape=jax.ShapeDtypeStruct((N, 128), F32),
    )(x, wab)


def _edge_tail_body(n_ea, *refs):
    xr, xc = refs[0], refs[1]
    eas = refs[2:2 + n_ea]
    wcs = refs[2 + n_ea:2 + 2 * n_ea]
    b1, w2, b2, w3, b3 = refs[2 + 2 * n_ea:7 + 2 * n_ea]
    out = refs[-1]
    h = xr[:, 0:64] + xc[:, 64:128] + b1[...]
    for ea, wc in zip(eas, wcs):
        h = h + _dot(ea[...], wc[...])
    h = _relu(h)
    h = _relu(_dot(h, w2[...]) + b2[...])
    out[...] = _dot(h, w3[...]) + b3[...]


def _tc_edge_tail(xr, xc, eas, wcs, b1, w2, b2, w3, b3, block_e=8000):
    E = xr.shape[0]
    n_ea = len(eas)
    grid = (E // block_e,)
    eb = lambda i: (i, 0)
    zb = lambda i: (0, 0)
    in_specs = (
        [pl.BlockSpec((block_e, 128), eb), pl.BlockSpec((block_e, 128), eb)]
        + [pl.BlockSpec((block_e, ea.shape[1]), eb) for ea in eas]
        + [pl.BlockSpec(w.shape, zb) for w in wcs]
        + [pl.BlockSpec(b1.shape, zb), pl.BlockSpec(w2.shape, zb),
           pl.BlockSpec(b2.shape, zb), pl.BlockSpec(w3.shape, zb),
           pl.BlockSpec(b3.shape, zb)]
    )
    return pl.pallas_call(
        functools.partial(_edge_tail_body, n_ea),
        grid=grid,
        in_specs=in_specs,
        out_specs=pl.BlockSpec((block_e, 16), eb),
        out_shape=jax.ShapeDtypeStruct((E, 16), F32),
    )(xr, xc, *eas, *wcs, b1, w2, b2, w3, b3)


def _node0_body(x_ref, mp_ref, batch_ref, a1x_ref, a1m_ref, a1b_ref,
                a2_ref, a2b_ref, a3_ref, a3b_ref,
                g1n_ref, g1e_ref, g1b_ref, g2_ref, g2b_ref, g3_ref, g3b_ref,
                vd1_ref, b1u_ref, wu_ref, wx_ref,
                updx_ref, t1_ref, un1_ref):
    half = mp_ref.shape[1] - 8
    msg = jnp.concatenate([mp_ref[0][0:half, 0:16], mp_ref[1][0:half, 0:16]], axis=0)
    x = x_ref[...]
    h = _relu(_dot(x, a1x_ref[...]) + _dot(msg, a1m_ref[...]) + a1b_ref[...])
    h = _relu(_dot(h, a2_ref[...]) + a2b_ref[...])
    updx = _dot(h, a3_ref[...]) + a3b_ref[...]
    updx_ref[...] = updx
    oh = (batch_ref[...] == lax.broadcasted_iota(jnp.int32, (1, 8), 1)).astype(F32)
    agg_n = _dot(oh.T, updx)
    agg_e = _dot(oh.T, msg)
    g = _relu(_dot(agg_n, g1n_ref[...]) + _dot(agg_e, g1e_ref[...]) + g1b_ref[...])
    g = _relu(_dot(g, g2_ref[...]) + g2b_ref[...])
    updu = _dot(g, g3_ref[...]) + g3b_ref[...]
    un1_ref[...] = _dot(updu, b1u_ref[...])
    ue1p = _dot(updu, vd1_ref[...])
    t1_ref[...] = (_dot(updx, wu_ref[...]) + _dot(x, wx_ref[...])
                   + _dot(oh, ue1p))


def _tc_node0(x, mp, batch2, node_w, glob_w, vd1, b1u, wu, wx):
    N = x.shape[0]
    (a1x, a1m, a1b), (a2, a2b), (a3, a3b) = node_w
    (g1n, g1e, g1b), (g2, g2b), (g3, g3b) = glob_w
    return pl.pallas_call(
        _node0_body,
        out_shape=(jax.ShapeDtypeStruct((N, 128), F32),
                   jax.ShapeDtypeStruct((N, 128), F32),
                   jax.ShapeDtypeStruct((8, 64), F32)),
    )(x, mp, batch2, a1x, a1m, a1b, a2, a2b, a3, a3b,
      g1n, g1e, g1b, g2, g2b, g3, g3b, vd1, b1u, wu, wx)


def _pre1_body(ux_ref, x_ref, batch_ref, ue1p_ref, wu_ref, wx_ref, t_ref):
    oh = (batch_ref[...] == lax.broadcasted_iota(jnp.int32, (1, 8), 1)).astype(F32)
    t_ref[...] = (_dot(ux_ref[...], wu_ref[...]) + _dot(x_ref[...], wx_ref[...])
                  + _dot(oh, ue1p_ref[...]))


def _tc_pre1(ux, x, batch2, ue1p, wu, wx):
    N = x.shape[0]
    return pl.pallas_call(
        _pre1_body,
        out_shape=jax.ShapeDtypeStruct((N, 128), F32),
    )(ux, x, batch2, ue1p, wu, wx)


def _node1_body(ux_ref, x_ref, mp_ref, batch_ref, un1_ref,
                b1a_ref, b1b_ref, b1m_ref, b1bias_ref,
                b2_ref, b2b_ref, b3_ref, b3b_ref,
                wo1_ref, wo2_ref, wo3_ref, bo_ref, out_ref):
    half = mp_ref.shape[1] - 8
    msg = jnp.concatenate([mp_ref[0][0:half, 0:16], mp_ref[1][0:half, 0:16]], axis=0)
    ux = ux_ref[...]
    x = x_ref[...]
    oh = (batch_ref[...] == lax.broadcasted_iota(jnp.int32, (1, 8), 1)).astype(F32)
    h = _relu(_dot(ux, b1a_ref[...]) + _dot(x, b1b_ref[...])
              + _dot(msg, b1m_ref[...]) + _dot(oh, un1_ref[...]) + b1bias_ref[...])
    h = _relu(_dot(h, b2_ref[...]) + b2b_ref[...])
    updx1 = _dot(h, b3_ref[...]) + b3b_ref[...]
    out_ref[...] = (_dot(updx1, wo1_ref[...]) + _dot(ux, wo2_ref[...])
                    + _dot(x, wo3_ref[...]) + bo_ref[...])


def _tc_node1_out(ux, x, mp, batch2, un1, node_w, wo1, wo2, wo3, bo):
    N = x.shape[0]
    (b1a, b1b, b1m, b1bias), (b2, b2b), (b3, b3b) = node_w
    return pl.pallas_call(
        _node1_body,
        out_shape=jax.ShapeDtypeStruct((N, 128), F32),
    )(ux, x, mp, batch2, un1, b1a, b1b, b1m, b1bias,
      b2, b2b, b3, b3b, wo1, wo2, wo3, bo)


def kernel(x, edge_index, edge_attr, batch, params):
    N = x.shape[0]
    E = edge_index.shape[1]
    x0 = x.reshape(N, x.shape[-1])
    ea0 = edge_attr.reshape(E, edge_attr.shape[-1])
    row3 = edge_index[0].reshape(E // _CH, _KI, _LANE)
    col3 = edge_index[1].reshape(E // _CH, _KI, _LANE)
    batch2 = batch.reshape(N, 1)
    col3e = edge_index[1].reshape(E // _CH, _KI, _LANE)
    zeros = jnp.zeros((N // 2 + 8, 128), F32)

    def row2(v):
        return v.reshape(1, -1)

    (W1, b1), (W2, b2), (W3, b3) = params['edge0']
    W1a, W1b, W1c = W1[:128], W1[128:256], W1[256:272]
    (A1, a1b), (A2, a2b), (A3, a3b) = params['node0']
    A1x, A1m = A1[:128], A1[128:144]
    (G1, g1b), (G2, g2b), (G3, g3b) = params['glob0']
    G1n, G1e = G1[:128], G1[128:144]
    (V1, v1b), (V2, v2b), (V3, v3b) = params['edge1']
    Va1, Va2 = V1[0:128], V1[128:256]
    Vb1, Vb2 = V1[256:384], V1[384:512]
    Vc1, Vc2 = V1[512:528], V1[528:544]
    Vd1 = V1[544:560]
    (B1, b1nb), (B2, b2nb), (B3, b3nb) = params['node1']
    B1a, B1b, B1m, B1u = B1[0:128], B1[128:256], B1[256:272], B1[272:288]
    Wo, bo = params['out'][0]
    Wo1, Wo2, Wo3 = Wo[0:128], Wo[128:256], Wo[256:384]

    T0 = _tc_pre0(x0, jnp.concatenate([W1a, W1b], axis=1))
    Xr0, Xc0 = _sc_gather(T0, row3, col3, E)
    upd_e0 = _tc_edge_tail(Xr0, Xc0, [ea0], [W1c],
                           row2(b1), W2, row2(b2), W3, row2(b3))
    mp0 = _sc_scatter(upd_e0.reshape(E // 8, 128), col3e, zeros, N, E)
    upd_x0, T1, Un1 = _tc_node0(
        x0, mp0, batch2,
        ((A1x, A1m, row2(a1b)), (A2, row2(a2b)), (A3, row2(a3b))),
        ((G1n, G1e, row2(g1b)), (G2, row2(g2b)), (G3, row2(g3b))),
        jnp.pad(Vd1, ((0, 0), (0, 64))), B1u,
        jnp.concatenate([Va1, Vb1], axis=1),
        jnp.concatenate([Va2, Vb2], axis=1))

    Xr1, Xc1 = _sc_gather(T1, row3, col3, E)
    upd_e1 = _tc_edge_tail(Xr1, Xc1, [upd_e0, ea0], [Vc1, Vc2],
                           row2(v1b), V2, row2(v2b), V3, row2(v3b))
    mp1 = _sc_scatter(upd_e1.reshape(E // 8, 128), col3e, zeros, N, E)
    out = _tc_node1_out(
        upd_x0, x0, mp1, batch2, Un1,
        ((B1a, B1b, B1m, row2(b1nb)), (B2, row2(b2nb)), (B3, row2(b3nb))),
        Wo1, Wo2, Wo3, row2(bo))

    return out.reshape(N, 1, 1, 128)
